# baseline (speedup 1.0000x reference)
import functools

import jax
import jax.numpy as jnp
from jax import lax
from jax.experimental import pallas as pl
from jax.experimental.pallas import tpu as pltpu
from jax.experimental.pallas import tpu_sc as plsc

S = 512
D = 512
NC = 2
NS = 16
NW = NC * NS
T = 16
BJ = 128
WROWS = 8 * (T - 1) + BJ
NITEM = 4

_mesh = plsc.VectorSubcoreMesh(core_axis_name="c", subcore_axis_name="s")


@functools.partial(
    pl.kernel,
    mesh=_mesh,
    out_type=jax.ShapeDtypeStruct((S * S, D), jnp.float32),
    scratch_types=[
        pltpu.VMEM((2 * T * 16,), jnp.int32),
        pltpu.VMEM((WROWS, D), jnp.float32),
        pltpu.SemaphoreType.DMA,
        pltpu.SemaphoreType.DMA,
        pltpu.SemaphoreType.DMA,
        pltpu.SemaphoreType.DMA,
    ],
)
def _expand(tab_hbm, out_hbm, idx_v, win_v, sem_r1, sem_r2, sem_r3, sem_w):
    wid = lax.axis_index("s") * NC + lax.axis_index("c")
    lane = lax.broadcasted_iota(jnp.int32, (16,), 0)
    top0 = (tab_hbm.shape[0] + 1) // 2 - 1 + (S - 1)

    for cc in range(NITEM):
        combo = cc * NW + wid
        m = combo % 8
        p = (combo // 8) % 4
        jb = combo // 32
        ibase = m + 128 * p
        j0 = jb * BJ
        w0 = (S - 1) - (ibase + 8 * (T - 1)) + j0
        top = top0 - w0
        for u in range(WROWS // 16 + 1):
            idx_v[pl.ds(u * 16, 16)] = (top - u * 16) - lane
        pieces = ((0, 128, sem_r1), (128, 48, sem_r2), (176, 72, sem_r3))
        for off, ln, sem in pieces:
            pltpu.make_async_copy(
                tab_hbm.at[idx_v.at[pl.ds(off, ln)]],
                win_v.at[pl.ds(off, ln)], sem).start()

        def fire(t):
            pltpu.make_async_copy(
                win_v.at[pl.ds(8 * (T - 1 - t), BJ)],
                out_hbm.at[pl.ds((ibase + 8 * t) * S + j0, BJ)],
                sem_w).start()

        unlocked = (range(T - 1, T), range(9, T - 1), range(0, 9))
        for (off, ln, sem), ts in zip(pieces, unlocked):
            pltpu.make_async_copy(
                tab_hbm.at[idx_v.at[pl.ds(off, ln)]],
                win_v.at[pl.ds(off, ln)], sem).wait()
            for t in ts:
                fire(t)
        for _ in range(T):
            pltpu.make_async_copy(
                win_v.at[pl.ds(0, BJ)],
                out_hbm.at[pl.ds(0, BJ)], sem_w).wait()


def kernel(seq_len, rel_pos_emb):
    del seq_len
    out_flat = _expand(rel_pos_emb)
    return out_flat.reshape(S, S, D)


# speedup vs baseline: 1.0098x; 1.0098x over previous
import functools

import jax
import jax.numpy as jnp
from jax import lax
from jax.experimental import pallas as pl
from jax.experimental.pallas import tpu as pltpu
from jax.experimental.pallas import tpu_sc as plsc

S = 512
D = 512
NC = 2
NS = 16
NW = NC * NS
T = 16
BJ = 128
WROWS = 8 * (T - 1) + BJ
NITEM = 4

_mesh = plsc.VectorSubcoreMesh(core_axis_name="c", subcore_axis_name="s")


@functools.partial(
    pl.kernel,
    mesh=_mesh,
    out_type=jax.ShapeDtypeStruct((S * S, D), jnp.float32),
    scratch_types=[
        pltpu.VMEM((2 * T * 16,), jnp.int32),
        pltpu.VMEM((WROWS, D), jnp.float32),
        pltpu.SemaphoreType.DMA,
        pltpu.SemaphoreType.DMA,
        pltpu.SemaphoreType.DMA,
        pltpu.SemaphoreType.DMA,
    ],
)
def _expand(tab_hbm, out_hbm, idx_v, win_v, sem_r1, sem_r2, sem_r3, sem_w):
    wid = lax.axis_index("s") * NC + lax.axis_index("c")
    lane = lax.broadcasted_iota(jnp.int32, (16,), 0)
    top0 = (tab_hbm.shape[0] + 1) // 2 - 1 + (S - 1)

    for cc in range(NITEM):
        combo = wid * NITEM + cc
        m = combo % 8
        p = (combo // 8) % 4
        jb = combo // 32
        ibase = m + 128 * p
        j0 = jb * BJ
        w0 = (S - 1) - (ibase + 8 * (T - 1)) + j0
        top = top0 - w0
        for u in range(WROWS // 16 + 1):
            idx_v[pl.ds(u * 16, 16)] = (top - u * 16) - lane
        pieces = ((0, 128, sem_r1), (128, 48, sem_r2), (176, 72, sem_r3))
        for off, ln, sem in pieces:
            pltpu.make_async_copy(
                tab_hbm.at[idx_v.at[pl.ds(off, ln)]],
                win_v.at[pl.ds(off, ln)], sem).start()

        def fire(t):
            pltpu.make_async_copy(
                win_v.at[pl.ds(8 * (T - 1 - t), BJ)],
                out_hbm.at[pl.ds((ibase + 8 * t) * S + j0, BJ)],
                sem_w).start()

        unlocked = (range(T - 1, T), range(9, T - 1), range(0, 9))
        for (off, ln, sem), ts in zip(pieces, unlocked):
            pltpu.make_async_copy(
                tab_hbm.at[idx_v.at[pl.ds(off, ln)]],
                win_v.at[pl.ds(off, ln)], sem).wait()
            for t in ts:
                fire(t)
        for _ in range(T):
            pltpu.make_async_copy(
                win_v.at[pl.ds(0, BJ)],
                out_hbm.at[pl.ds(0, BJ)], sem_w).wait()


def kernel(seq_len, rel_pos_emb):
    del seq_len
    out_flat = _expand(rel_pos_emb)
    return out_flat.reshape(S, S, D)
